# Initial kernel scaffold; baseline (speedup 1.0000x reference)
import jax
import jax.numpy as jnp
from jax.experimental import pallas as pl


def kernel(token_ids, emb_table, pos_table):
    raise NotImplementedError("write your pallas kernel here")


# speedup vs baseline: 2.1946x; 2.1946x over previous
import functools

import jax
import jax.numpy as jnp
from jax import lax
from jax.experimental import pallas as pl
from jax.experimental.pallas import tpu as pltpu
from jax.experimental.pallas import tpu_sc as plsc

_B, _S, _D = 4, 2048, 1024
_PAD = 0

_info = plsc.get_sparse_core_info()
_NC, _NS, _L = _info.num_cores, _info.num_subcores, _info.num_lanes
_NW = _NC * _NS
_PPW = _S // _NW
_CH = 32
_NCHUNK = _PPW // _CH
_NCOL = _D // _L

_mesh = plsc.VectorSubcoreMesh(core_axis_name="c", subcore_axis_name="s")


@functools.partial(
    pl.kernel,
    mesh=_mesh,
    out_type=jax.ShapeDtypeStruct((_B * _S, _D), jnp.float32),
    scratch_types=[
        pltpu.VMEM((_B, _PPW), jnp.int32),
        pltpu.VMEM((_PPW, _D), jnp.float32),
        pltpu.VMEM((_CH, _D), jnp.float32),
        pltpu.SemaphoreType.DMA,
    ],
)
def _emb_kernel(tok_hbm, emb_hbm, pos_hbm, out_hbm, idx_v, pos_v, row_v, sem):
    wid = lax.axis_index("s") * _NC + lax.axis_index("c")
    p0 = wid * _PPW

    pltpu.sync_copy(pos_hbm.at[pl.ds(p0, _PPW), :], pos_v)
    for b in range(_B):
        pltpu.sync_copy(tok_hbm.at[b, pl.ds(p0, _PPW)], idx_v.at[b])

    for b in range(_B):
        for ch in range(_NCHUNK):
            idx_c = idx_v.at[b, pl.ds(ch * _CH, _CH)]
            pltpu.async_copy(emb_hbm.at[idx_c], row_v, sem).wait()

            for g in range(_CH // _L):
                tok16 = idx_v[b, pl.ds(ch * _CH + g * _L, _L)]
                mask16 = jnp.where(tok16 == _PAD, 0.0, 1.0).astype(jnp.float32)

                def tok_body(j, _, g=g, ch=ch, mask16=mask16):
                    mv = mask16.at[lax.broadcast(j, (_L,))].get(
                        mode="promise_in_bounds")
                    r = g * _L + j

                    def col_body(c, _):
                        e = row_v[r, pl.ds(c * _L, _L)]
                        p = pos_v[ch * _CH + r, pl.ds(c * _L, _L)]
                        row_v[r, pl.ds(c * _L, _L)] = e * mv + p
                        return 0

                    lax.fori_loop(0, _NCOL, col_body, 0)
                    return 0

                lax.fori_loop(0, _L, tok_body, 0)

            base = b * _S + p0 + ch * _CH
            pltpu.sync_copy(row_v, out_hbm.at[pl.ds(base, _CH), :])


def kernel(token_ids, emb_table, pos_table):
    out = _emb_kernel(token_ids, emb_table, pos_table)
    return out.reshape(_B, _S, _D)


# speedup vs baseline: 2.2308x; 1.0165x over previous
import functools

import jax
import jax.numpy as jnp
from jax import lax
from jax.experimental import pallas as pl
from jax.experimental.pallas import tpu as pltpu
from jax.experimental.pallas import tpu_sc as plsc

_B, _S, _D = 4, 2048, 1024
_PAD = 0

_info = plsc.get_sparse_core_info()
_NC, _NS, _L = _info.num_cores, _info.num_subcores, _info.num_lanes
_NW = _NC * _NS
_PPW = _S // _NW
_CH = 32
_NSTEP = _B * (_PPW // _CH)
_UN = 8
_NCOL = _D // _L

_mesh = plsc.VectorSubcoreMesh(core_axis_name="c", subcore_axis_name="s")


@functools.partial(
    pl.kernel,
    mesh=_mesh,
    out_type=jax.ShapeDtypeStruct((_B * _S, _D), jnp.float32),
    scratch_types=[
        pltpu.VMEM((_B, _PPW), jnp.int32),
        pltpu.VMEM((_CH, _D), jnp.float32),
        pltpu.VMEM((_CH, _D), jnp.float32),
        pltpu.VMEM((_CH, _D), jnp.float32),
        pltpu.SemaphoreType.DMA,
        pltpu.SemaphoreType.DMA,
        pltpu.SemaphoreType.DMA,
        pltpu.SemaphoreType.DMA,
    ],
)
def _emb_kernel(tok_hbm, emb_hbm, pos_hbm, out_hbm,
                idx_v, pos_v, row_a, row_b, gsem_a, gsem_b, wsem_a, wsem_b):
    wid = lax.axis_index("s") * _NC + lax.axis_index("c")
    p0 = wid * _PPW

    rows = (row_a, row_b)
    gsems = (gsem_a, gsem_b)
    wsems = (wsem_a, wsem_b)

    for b in range(_B):
        pltpu.sync_copy(tok_hbm.at[b, pl.ds(p0, _PPW)], idx_v.at[b])

    def start_gather(k, buf):
        h, b = k // _B, k % _B
        idx_c = idx_v.at[b, pl.ds(h * _CH, _CH)]
        return pltpu.async_copy(emb_hbm.at[idx_c], rows[buf], gsems[buf])

    def start_write(k, buf):
        h, b = k // _B, k % _B
        base = b * _S + p0 + h * _CH
        return pltpu.async_copy(rows[buf], out_hbm.at[pl.ds(base, _CH), :], wsems[buf])

    def compute(k, buf):
        h, b = k // _B, k % _B
        buf_ref = rows[buf]
        for g in range(_CH // _L):
            tok16 = idx_v[b, pl.ds(h * _CH + g * _L, _L)]
            mask16 = jnp.where(tok16 == _PAD, 0.0, 1.0).astype(jnp.float32)

            def tok_body(j, _, g=g, mask16=mask16):
                mv = mask16.at[lax.broadcast(j, (_L,))].get(
                    mode="promise_in_bounds")
                r = g * _L + j

                def col_body(c, _):
                    for u in range(_UN):
                        off = c * (_UN * _L) + u * _L
                        e = buf_ref[r, pl.ds(off, _L)]
                        p = pos_v[r, pl.ds(off, _L)]
                        buf_ref[r, pl.ds(off, _L)] = e * mv + p
                    return 0

                lax.fori_loop(0, _NCOL // _UN, col_body, 0)
                return 0

            lax.fori_loop(0, _L, tok_body, 0)

    pltpu.sync_copy(pos_hbm.at[pl.ds(p0, _CH), :], pos_v)
    g = [start_gather(0, 0), None]
    w = [None, None]
    for k in range(_NSTEP):
        cur = k % 2
        nxt = 1 - cur
        if k + 1 < _NSTEP:
            if w[nxt] is not None:
                w[nxt].wait()
            g[nxt] = start_gather(k + 1, nxt)
        if k == _B:
            pltpu.sync_copy(pos_hbm.at[pl.ds(p0 + _CH, _CH), :], pos_v)
        g[cur].wait()
        compute(k, cur)
        w[cur] = start_write(k, cur)
    w[0].wait()
    w[1].wait()


def kernel(token_ids, emb_table, pos_table):
    out = _emb_kernel(token_ids, emb_table, pos_table)
    return out.reshape(_B, _S, _D)
